# baseline (speedup 1.0000x reference)
import jax
import jax.numpy as jnp
from jax import lax
from jax.experimental import pallas as pl
from jax.experimental.pallas import tpu as pltpu
from jax.experimental.pallas import tpu_sc as plsc

_NC, _NS, _L = 2, 16, 16
_NW = _NC * _NS
_D = 2048
_K = 8

_TC_BLOCK = 512
_SC_FRAC_NUM = 3
_SC_FRAC_DEN = 8


def _splat(vec, lane):
    idx = jnp.full((_L,), lane, jnp.int32)
    return lax.gather(
        vec, idx[:, None],
        dimension_numbers=lax.GatherDimensionNumbers(
            offset_dims=(), collapsed_slice_dims=(0,), start_index_map=(0,)),
        slice_sizes=(1,),
        mode=lax.GatherScatterMode.PROMISE_IN_BOUNDS)


def _tc_body(p_ref, t_ref, m_ref, num_ref, den_ref):
    i = pl.program_id(0)

    @pl.when(i == 0)
    def _init():
        num_ref[0, 0] = 0.0
        den_ref[0, 0] = 0.0

    diff = p_ref[...] - t_ref[...]
    row_sq = jnp.sum(diff * diff, axis=1, keepdims=True)
    m = (m_ref[...] != 0).astype(jnp.float32)
    num_ref[0, 0] += jnp.sum(row_sq * m)
    den_ref[0, 0] += jnp.sum(m)


def _tc_partial(p, t, m2, rows):
    grid = (rows // _TC_BLOCK,)
    return pl.pallas_call(
        _tc_body,
        grid=grid,
        cost_estimate=pl.CostEstimate(
            flops=3 * rows * _D, transcendentals=0,
            bytes_accessed=rows * _D * 4 * 2),
        in_specs=[
            pl.BlockSpec((_TC_BLOCK, _D), lambda i: (i, 0)),
            pl.BlockSpec((_TC_BLOCK, _D), lambda i: (i, 0)),
            pl.BlockSpec((_TC_BLOCK, 1), lambda i: (i, 0)),
        ],
        out_specs=[
            pl.BlockSpec(memory_space=pltpu.SMEM),
            pl.BlockSpec(memory_space=pltpu.SMEM),
        ],
        out_shape=[
            jax.ShapeDtypeStruct((1, 1), jnp.float32),
            jax.ShapeDtypeStruct((1, 1), jnp.float32),
        ],
    )(p, t, m2)


def _make_sc_dense(total_rows, sc_base):
    sc_rows = total_rows - sc_base
    assert sc_rows % (_NW * _K) == 0
    rpw = sc_rows // _NW
    n_chunks = rpw // _K
    assert n_chunks % 2 == 0

    def body(p_hbm, t_hbm, m_hbm, num_hbm, den_hbm,
             mask_v, p_buf, t_buf, acc_v, sem_p0, sem_t0, sem_p1, sem_t1):
        wid = lax.axis_index("s") * _NC + lax.axis_index("c")
        base = sc_base + wid * rpw

        pltpu.sync_copy(m_hbm.at[pl.ds(base, rpw)], mask_v.at[pl.ds(0, rpw)])

        zf = jnp.zeros((_L,), jnp.float32)
        lane_lo = lax.iota(jnp.int32, _L) < _K

        sems = ((sem_p0, sem_t0), (sem_p1, sem_t1))

        def start(j, slot):
            row0 = base + j * _K
            sp, st = sems[slot]
            pltpu.make_async_copy(
                p_hbm.at[pl.ds(row0, _K)], p_buf.at[slot], sp).start()
            pltpu.make_async_copy(
                t_hbm.at[pl.ds(row0, _K)], t_buf.at[slot], st).start()

        def wait(slot):
            sp, st = sems[slot]
            pltpu.make_async_copy(
                p_hbm.at[pl.ds(base, _K)], p_buf.at[slot], sp).wait()
            pltpu.make_async_copy(
                t_hbm.at[pl.ds(base, _K)], t_buf.at[slot], st).wait()

        def compute(j, slot, accs):
            acc, den = accs
            wvec = mask_v[pl.ds(j * _K, _L)]
            wf = jnp.where(wvec != jnp.int32(0), 1.0, 0.0).astype(jnp.float32)
            den = den + jnp.where(lane_lo, wf, 0.0)
            for r in range(_K):
                def col_body(c, racc):
                    o = c * (_L * 4)
                    out = racc
                    for u in range(4):
                        dlt = (p_buf[slot, r, pl.ds(o + u * _L, _L)]
                               - t_buf[slot, r, pl.ds(o + u * _L, _L)])
                        out = out + dlt * dlt
                    return out
                racc = lax.fori_loop(0, _D // (_L * 4), col_body, zf)
                acc = acc + racc * _splat(wf, r)
            return acc, den

        start(0, 0)
        start(1, 1)

        def pair_body(jj, accs):
            j = jj * 2
            wait(0)
            accs = compute(j, 0, accs)

            @pl.when(j + 2 < n_chunks)
            def _():
                start(j + 2, 0)
            wait(1)
            accs = compute(j + 1, 1, accs)

            @pl.when(j + 3 < n_chunks)
            def _():
                start(j + 3, 1)
            return accs

        acc, den = lax.fori_loop(0, n_chunks // 2, pair_body, (zf, zf))

        acc_v[...] = acc
        pltpu.sync_copy(acc_v, num_hbm.at[wid])
        acc_v[...] = den
        pltpu.sync_copy(acc_v, den_hbm.at[wid])

    mesh = plsc.VectorSubcoreMesh(core_axis_name="c", subcore_axis_name="s")
    sc_bytes = sc_rows * _D * 4 * 2 + sc_rows * 4
    return pl.kernel(
        body,
        mesh=mesh,
        cost_estimate=pl.CostEstimate(
            flops=3 * sc_rows * _D, transcendentals=0,
            bytes_accessed=sc_bytes),
        out_type=[
            jax.ShapeDtypeStruct((_NW, _L), jnp.float32),
            jax.ShapeDtypeStruct((_NW, _L), jnp.float32),
        ],
        scratch_types=[
            pltpu.VMEM((rpw + _L,), jnp.int32),
            pltpu.VMEM((2, _K, _D), jnp.float32),
            pltpu.VMEM((2, _K, _D), jnp.float32),
            pltpu.VMEM((_L,), jnp.float32),
            pltpu.SemaphoreType.DMA,
            pltpu.SemaphoreType.DMA,
            pltpu.SemaphoreType.DMA,
            pltpu.SemaphoreType.DMA,
        ],
    )


@jax.jit
def kernel(prediction, target, mask):
    d = prediction.shape[-1]
    p = prediction.reshape(-1, d)
    t = target.reshape(-1, d)
    n = p.shape[0]
    m = mask.reshape(-1).astype(jnp.int32)

    tc_rows = (n * (_SC_FRAC_DEN - _SC_FRAC_NUM) // _SC_FRAC_DEN)
    tc_rows = (tc_rows // _TC_BLOCK) * _TC_BLOCK

    tc_num, tc_den = _tc_partial(p, t, m.reshape(-1, 1), tc_rows)

    sc_num, sc_den = _make_sc_dense(n, tc_rows)(p, t, m)

    num = tc_num[0, 0] + jnp.sum(sc_num)
    den = tc_den[0, 0] + jnp.sum(sc_den)
    return num / (den * d)


# speedup vs baseline: 1.0661x; 1.0661x over previous
import jax
import jax.numpy as jnp
from hybrid import _make_sc_dense, _TC_BLOCK, _SC_FRAC_NUM, _SC_FRAC_DEN


@jax.jit
def kernel(prediction, target, mask):
    d = prediction.shape[-1]
    p = prediction.reshape(-1, d)
    t = target.reshape(-1, d)
    n = p.shape[0]
    m = mask.reshape(-1).astype(jnp.int32)

    tc_rows = (n * (_SC_FRAC_DEN - _SC_FRAC_NUM) // _SC_FRAC_DEN)
    tc_rows = (tc_rows // _TC_BLOCK) * _TC_BLOCK

    sc_num, sc_den = _make_sc_dense(n, tc_rows)(p, t, m)

    mf = (m[:tc_rows] != 0).astype(jnp.float32)
    diff = p[:tc_rows] - t[:tc_rows]
    xla_num = jnp.sum(jnp.sum(diff * diff, axis=1) * mf)
    xla_den = jnp.sum(mf)

    num = xla_num + jnp.sum(sc_num)
    den = xla_den + jnp.sum(sc_den)
    return num / (den * d)


# speedup vs baseline: 1.1473x; 1.0762x over previous
import functools

import jax
import jax.numpy as jnp
from jax.experimental import pallas as pl
from jax.experimental.pallas import tpu as pltpu

_BLOCK_ROWS = 512


def _loss_body(p_ref, t_ref, m_ref, num_ref, den_ref):
    i = pl.program_id(0)

    @pl.when(i == 0)
    def _init():
        num_ref[0, 0] = 0.0
        den_ref[0, 0] = 0.0

    diff = p_ref[...] - t_ref[...]
    row_sq = jnp.sum(diff * diff, axis=1, keepdims=True)
    m = (m_ref[...] != 0).astype(jnp.float32)
    num_ref[0, 0] += jnp.sum(row_sq * m)
    den_ref[0, 0] += jnp.sum(m)


@jax.jit
def kernel(prediction, target, mask):
    d = prediction.shape[-1]
    p = prediction.reshape(-1, d)
    t = target.reshape(-1, d)
    n = p.shape[0]
    m = mask.reshape(-1, 1).astype(jnp.int32)

    rows = min(_BLOCK_ROWS, n)
    grid = (n // rows,)

    num, den = pl.pallas_call(
        _loss_body,
        grid=grid,
        in_specs=[
            pl.BlockSpec((rows, d), lambda i: (i, 0)),
            pl.BlockSpec((rows, d), lambda i: (i, 0)),
            pl.BlockSpec((rows, 1), lambda i: (i, 0)),
        ],
        out_specs=[
            pl.BlockSpec(memory_space=pltpu.SMEM),
            pl.BlockSpec(memory_space=pltpu.SMEM),
        ],
        out_shape=[
            jax.ShapeDtypeStruct((1, 1), jnp.float32),
            jax.ShapeDtypeStruct((1, 1), jnp.float32),
        ],
    )(p, t, m)

    return num[0, 0] / (den[0, 0] * d)


# speedup vs baseline: 1.1502x; 1.0025x over previous
import jax
import jax.numpy as jnp
from jax.experimental import pallas as pl
from jax.experimental.pallas import tpu as pltpu

_BLOCK_ROWS = 512


def _loss_body(p_ref, t_ref, m_ref, num_ref, den_ref):
    i = pl.program_id(0)

    @pl.when(i == 0)
    def _init():
        num_ref[0, 0] = 0.0
        den_ref[0, 0] = 0.0

    rows = p_ref.shape[0]
    diff = p_ref[...] - t_ref[...]
    row_sq = jnp.sum(diff * diff, axis=1, keepdims=True)
    m = (m_ref[pl.ds(i * rows, rows), :] != 0).astype(jnp.float32)
    num_ref[0, 0] += jnp.sum(row_sq * m)
    den_ref[0, 0] += jnp.sum(m)


@jax.jit
def kernel(prediction, target, mask):
    d = prediction.shape[-1]
    p = prediction.reshape(-1, d)
    t = target.reshape(-1, d)
    n = p.shape[0]
    m = mask.reshape(-1, 1).astype(jnp.int32)

    rows = min(_BLOCK_ROWS, n)
    grid = (n // rows,)

    num, den = pl.pallas_call(
        _loss_body,
        grid=grid,
        in_specs=[
            pl.BlockSpec((rows, d), lambda i: (i, 0)),
            pl.BlockSpec((rows, d), lambda i: (i, 0)),
            pl.BlockSpec((n, 1), lambda i: (0, 0)),
        ],
        out_specs=[
            pl.BlockSpec(memory_space=pltpu.SMEM),
            pl.BlockSpec(memory_space=pltpu.SMEM),
        ],
        out_shape=[
            jax.ShapeDtypeStruct((1, 1), jnp.float32),
            jax.ShapeDtypeStruct((1, 1), jnp.float32),
        ],
    )(p, t, m)

    return num[0, 0] / (den[0, 0] * d)
